# baseline (speedup 1.0000x reference)
import functools

import jax
import jax.numpy as jnp
from jax.experimental import pallas as pl
import jax.experimental.pallas.tpu as pltpu


def _vq_body(zt_ref, cb_ref, o_ref, cbm2_ref, e2_ref, *, bk, n_kc, sl_rows,
             dot_rows):
    b = pl.program_id(0)
    w = zt_ref.shape[2]

    @pl.when(b == 0)
    def _build():
        for kc in range(n_kc):
            rows = slice(kc * bk, (kc + 1) * bk)
            cb = cb_ref[rows, :]
            cbm2_ref[rows, :] = cb * -2.0
            e2 = jnp.sum(cb * cb, axis=1, keepdims=True)
            e2_ref[rows, :] = jnp.broadcast_to(e2, (bk, w))

    nb = zt_ref.shape[0]
    lats = [zt_ref[i] for i in range(nb)]
    zzs = [jnp.sum(l * l, axis=0, keepdims=True) for l in lats]

    inf = jnp.full((sl_rows, w), jnp.inf, jnp.float32)
    zero = jnp.zeros((sl_rows, w), jnp.float32)
    rms = [inf] * nb
    sis = [zero] * nb

    k_total = cb_ref.shape[0]
    n_dots = k_total // dot_rows
    per_dot = dot_rows // sl_rows
    dn = (((1,), (0,)), ((), ()))
    for kd in range(n_dots):
        cbs = cbm2_ref[kd * dot_rows:(kd + 1) * dot_rows, :]
        sm2s = [jax.lax.dot_general(cbs, l, dn,
                                    preferred_element_type=jnp.float32)
                for l in lats]
        for sl in range(per_dot):
            rows = slice(sl * sl_rows, (sl + 1) * sl_rows)
            e2s = e2_ref[kd * dot_rows + sl * sl_rows:
                         kd * dot_rows + (sl + 1) * sl_rows, :]
            gs = jnp.float32(kd * per_dot + sl)
            for i in range(nb):
                d = (zzs[i] + e2s) + sm2s[i][rows, :]
                upd = d < rms[i]
                sis[i] = jnp.where(upd, gs, sis[i])
                rms[i] = jnp.minimum(rms[i], d)

    pos = jax.lax.broadcasted_iota(jnp.int32, (sl_rows, w), 0).astype(jnp.float32)
    for i, (rm, si) in enumerate(zip(rms, sis)):
        rows_g = si * jnp.float32(sl_rows) + pos
        gmin = jnp.min(rm, axis=0, keepdims=True)
        cand = jnp.where(rm == gmin, rows_g, jnp.inf)
        best = jnp.min(cand, axis=0, keepdims=True)
        o_ref[i] = best.astype(jnp.int32)


def kernel(latent, codebook):
    B, C, H, W = latent.shape
    K, _ = codebook.shape
    n_tok = H * W
    zt = latent.reshape(B, C, n_tok)

    BK = 1024
    n_kc = K // BK
    SL = 8
    DR = 128
    NB = 16

    out = pl.pallas_call(
        functools.partial(_vq_body, bk=BK, n_kc=n_kc, sl_rows=SL, dot_rows=DR),
        grid=(B // NB,),
        in_specs=[
            pl.BlockSpec((NB, C, n_tok), lambda b: (b, 0, 0)),
            pl.BlockSpec((K, C), lambda b: (0, 0)),
        ],
        out_specs=pl.BlockSpec((NB, 1, n_tok), lambda b: (b, 0, 0)),
        out_shape=jax.ShapeDtypeStruct((B, 1, n_tok), jnp.int32),
        scratch_shapes=[
            pltpu.VMEM((K, C), jnp.float32),
            pltpu.VMEM((K, n_tok), jnp.float32),
        ],
        compiler_params=pltpu.CompilerParams(
            dimension_semantics=("arbitrary",),
        ),
    )(zt, codebook)
    return out.reshape(B, n_tok)


# speedup vs baseline: 1.1965x; 1.1965x over previous
import functools

import jax
import jax.numpy as jnp
from jax.experimental import pallas as pl
import jax.experimental.pallas.tpu as pltpu


def _vq_body(zt_ref, cb_ref, o_ref, cbm2_ref, e2_ref, *, bk, n_kc, sl_rows,
             dot_rows):
    b = pl.program_id(0)
    w = zt_ref.shape[2]

    @pl.when(b == 0)
    def _build():
        for kc in range(n_kc):
            rows = slice(kc * bk, (kc + 1) * bk)
            cb = cb_ref[rows, :]
            cbm2_ref[rows, :] = cb * -2.0
            e2 = jnp.sum(cb * cb, axis=1, keepdims=True)
            e2_ref[rows, :] = jnp.broadcast_to(e2, (bk, w))

    nb = zt_ref.shape[0]
    lats = [zt_ref[i] for i in range(nb)]
    zzs = [jnp.sum(l * l, axis=0, keepdims=True) for l in lats]

    inf = jnp.full((sl_rows, w), jnp.inf, jnp.float32)
    zero = jnp.zeros((sl_rows, w), jnp.float32)
    rms = [inf] * nb
    sis = [zero] * nb

    k_total = cb_ref.shape[0]
    n_dots = k_total // dot_rows
    per_dot = dot_rows // sl_rows
    dn = (((1,), (0,)), ((), ()))
    for kd in range(n_dots):
        cbs = cbm2_ref[kd * dot_rows:(kd + 1) * dot_rows, :]
        sm2s = [jax.lax.dot_general(cbs, l, dn,
                                    preferred_element_type=jnp.float32)
                for l in lats]
        for sl in range(per_dot):
            rows = slice(sl * sl_rows, (sl + 1) * sl_rows)
            e2s = e2_ref[kd * dot_rows + sl * sl_rows:
                         kd * dot_rows + (sl + 1) * sl_rows, :]
            gs = jnp.float32(kd * per_dot + sl)
            for i in range(nb):
                d = (zzs[i] + e2s) + sm2s[i][rows, :]
                upd = d < rms[i]
                sis[i] = jnp.where(upd, gs, sis[i])
                rms[i] = jnp.minimum(rms[i], d)

    pos = jax.lax.broadcasted_iota(jnp.int32, (sl_rows, w), 0).astype(jnp.float32)
    for i, (rm, si) in enumerate(zip(rms, sis)):
        rows_g = si * jnp.float32(sl_rows) + pos
        gmin = jnp.min(rm, axis=0, keepdims=True)
        cand = jnp.where(rm == gmin, rows_g, jnp.inf)
        best = jnp.min(cand, axis=0, keepdims=True)
        o_ref[i] = best.astype(jnp.int32)


def kernel(latent, codebook):
    B, C, H, W = latent.shape
    K, _ = codebook.shape
    n_tok = H * W
    zt = latent.reshape(B, C, n_tok)

    BK = 1024
    n_kc = K // BK
    SL = 8
    DR = 128
    NB = 8

    out = pl.pallas_call(
        functools.partial(_vq_body, bk=BK, n_kc=n_kc, sl_rows=SL, dot_rows=DR),
        grid=(B // NB,),
        in_specs=[
            pl.BlockSpec((NB, C, n_tok), lambda b: (b, 0, 0)),
            pl.BlockSpec((K, C), lambda b: (0, 0)),
        ],
        out_specs=pl.BlockSpec((NB, 1, n_tok), lambda b: (b, 0, 0)),
        out_shape=jax.ShapeDtypeStruct((B, 1, n_tok), jnp.int32),
        scratch_shapes=[
            pltpu.VMEM((K, C), jnp.float32),
            pltpu.VMEM((K, n_tok), jnp.float32),
        ],
        compiler_params=pltpu.CompilerParams(
            dimension_semantics=("arbitrary",),
        ),
    )(zt, codebook)
    return out.reshape(B, n_tok)
